# baseline (speedup 1.0000x reference)
import functools

import jax
import jax.numpy as jnp
from jax import lax
from jax.experimental import pallas as pl
from jax.experimental.pallas import tpu as pltpu
from jax.experimental.pallas import tpu_sc as plsc

B, N, C = 8, 20000, 91
L = 16
NC, NS = 2, 16
NW = NC * NS
ST = 40
NSC = ST * 128
UPW = 2
CA, CB = 49, 42
OFFS = (0, 16, 32, 48, 64, 75)
ASTR = 96
BN = 2560
NTC = N - NSC


def _permute(g, idx):
  dn = lax.GatherDimensionNumbers(offset_dims=(), collapsed_slice_dims=(0,),
                                  start_index_map=(0,))
  return lax.gather(g, idx[:, None], dn, (1,),
                    mode=lax.GatherScatterMode.PROMISE_IN_BOUNDS)


def _sc_body(lg_hbm, pr_hbm, out_hbm,
             b0_v, b1_v, acc_v, sco_v, pr_v, at_v, semA, semB, semW):
  w = lax.axis_index("s") * NC + lax.axis_index("c")
  active = w * UPW < ST

  pltpu.sync_copy(pr_hbm.at[:, :], pr_v)

  def tab_img(img, carry):
    avecs = [1.0 + jnp.exp(-pr_v[img, pl.ds(off, L)]) for off in OFFS]

    def tab_lane(l, carry2):
      bl = jnp.broadcast_to(l, (L,))
      for j, off in enumerate(OFFS):
        at_v[pl.ds((img * ASTR + off + l) * L, L)] = _permute(avecs[j], bl)
      return carry2

    lax.fori_loop(0, L, tab_lane, 0)
    return carry

  lax.fori_loop(0, B, tab_img, 0)

  def issue(tc, buf, nclass, c0, sem):
    pltpu.async_copy(
        lg_hbm.at[pl.ds(c0, nclass), :, pl.ds(tc * 128, 128)], buf, sem)

  def wait_in(buf, nclass, sem):
    pltpu.make_async_copy(
        lg_hbm.at[pl.ds(0, nclass), :, pl.ds(0, 128)], buf, sem).wait()

  @pl.when(active)
  def _prologue():
    issue(w * UPW, b0_v, CA, 0, semA)
    issue(w * UPW, b1_v, CB, CA, semB)

  NA = 8
  inf8 = (jnp.full((L,), jnp.inf, jnp.float32),) * NA

  def make_cbody(buf, img, cbase):
    def cbody(c, accs):
      sp = at_v[pl.ds((img * ASTR + cbase + c) * L, L)]
      out = []
      for i in range(NA):
        x = buf[c, img, pl.ds(i * L, L)]
        out.append(jnp.minimum(accs[i], sp * jnp.exp(-x) + sp))
      return tuple(out)
    return cbody

  def compute_a(carry_unused):
    def grp(img, carry):
      accs = lax.fori_loop(0, CA, make_cbody(b0_v, img, 0), inf8, unroll=1)
      for i in range(NA):
        acc_v[pl.ds(img * 128 + i * L, L)] = accs[i]
      return carry
    lax.fori_loop(0, B, grp, 0)

  def compute_b(p):
    def grp(img, carry):
      init = tuple(acc_v[pl.ds(img * 128 + i * L, L)] for i in range(NA))
      accs = lax.fori_loop(0, CB, make_cbody(b1_v, img, CA), init, unroll=1)
      for i in range(NA):
        sco_v[p, img, pl.ds(i * L, L)] = 1.0 / accs[i]
      return carry
    lax.fori_loop(0, B, grp, 0)

  def ubody(k, carry):
    tc = w * UPW + k
    p = k & 1
    wait_in(b0_v, CA, semA)
    compute_a(None)

    @pl.when(k + 1 < UPW)
    def _ia():
      issue(tc + 1, b0_v, CA, 0, semA)

    wait_in(b1_v, CB, semB)
    compute_b(p)
    pltpu.async_copy(sco_v.at[p], out_hbm.at[:, pl.ds(tc * 128, 128)], semW)

    @pl.when(k + 1 < UPW)
    def _ib():
      issue(tc + 1, b1_v, CB, CA, semB)

    return carry

  @pl.when(active)
  def _run():
    lax.fori_loop(0, UPW, ubody, 0)
    for _ in range(UPW):
      pltpu.make_async_copy(sco_v.at[0], out_hbm.at[:, pl.ds(0, 128)],
                            semW).wait()


_sc_scores = functools.partial(
    pl.kernel,
    out_type=jax.ShapeDtypeStruct((B, NSC), jnp.float32),
    mesh=plsc.VectorSubcoreMesh(core_axis_name="c", subcore_axis_name="s",
                                num_cores=NC, num_subcores=NS),
    scratch_types=[
        pltpu.VMEM((CA, B, 128), jnp.float32),
        pltpu.VMEM((CB, B, 128), jnp.float32),
        pltpu.VMEM((B * 128,), jnp.float32),
        pltpu.VMEM((2, B, 128), jnp.float32),
        pltpu.VMEM((B, 128), jnp.float32),
        pltpu.VMEM((B * ASTR * L,), jnp.float32),
        pltpu.SemaphoreType.DMA,
        pltpu.SemaphoreType.DMA,
        pltpu.SemaphoreType.DMA,
    ],
    compiler_params=pltpu.CompilerParams(use_tc_tiling_on_sc=True))(_sc_body)


NG = NTC // BN + 1
BB = 128 * (-(-N // (128 * NG)))


def _tc_main_body(ts_ref, prt_ref, lg_ref, bx_ref,
                  sco_ref, box_ref, lab_ref):
  x = lg_ref[...]
  a = 1.0 + jnp.exp(-prt_ref[...][:C])
  acc = jnp.min(a[:, :, None] * jnp.exp(-x) + a[:, :, None], axis=0)
  sco_ref[...] = 1.0 / acc
  xb = bx_ref[...]
  ts = ts_ref[...].astype(jnp.float32)
  hh = ts[0][:, None, None]
  ww = ts[1][:, None, None]
  coord = lax.broadcasted_iota(jnp.int32, (B, 4, BB), 1)
  half = jnp.where(coord >= 2, 0.5, -0.5)
  cxy = jnp.concatenate([xb[:, 0:2], xb[:, 0:2]], axis=1)
  wh = jnp.concatenate([xb[:, 2:4], xb[:, 2:4]], axis=1)
  scale = jnp.where(coord % 2 == 0, ww, hh)
  box_ref[...] = (cxy + half * wh) * scale
  lab_ref[...] = jnp.ones((B, BB), jnp.int32)


def _tc_main(lgt, prt, bxt, tst):
  return pl.pallas_call(
      _tc_main_body,
      grid=(NG,),
      in_specs=[
          pl.BlockSpec((2, B), lambda j: (0, 0)),
          pl.BlockSpec((128, B), lambda j: (0, 0)),
          pl.BlockSpec((C, B, BN), lambda j: (0, 0, j + NSC // BN)),
          pl.BlockSpec((B, 4, BB), lambda j: (0, 0, j)),
      ],
      out_specs=[
          pl.BlockSpec((B, BN), lambda j: (0, j)),
          pl.BlockSpec((B, 4, BB), lambda j: (0, 0, j)),
          pl.BlockSpec((B, BB), lambda j: (0, j)),
      ],
      out_shape=[
          jax.ShapeDtypeStruct((B, NTC), jnp.float32),
          jax.ShapeDtypeStruct((B, 4, N), jnp.float32),
          jax.ShapeDtypeStruct((B, N), jnp.int32),
      ],
  )(tst, prt, lgt, bxt)


def kernel(pred_logits, pred_boxes, presence_logit_dec,
           target_sizes_boxes, target_sizes_masks):
  del target_sizes_masks
  lgt = jnp.transpose(pred_logits, (2, 0, 1))
  bxt = jnp.transpose(pred_boxes, (0, 2, 1))
  tst = jnp.transpose(target_sizes_boxes, (1, 0))
  pr_pad = jnp.pad(presence_logit_dec, ((0, 0), (0, 128 - C)))
  prt = jnp.transpose(pr_pad, (1, 0))
  sc_part = _sc_scores(lgt, pr_pad)
  tc_part, boxes_t, labels = _tc_main(lgt, prt, bxt, tst)
  scores = jnp.concatenate([sc_part, tc_part], axis=1)
  boxes = jnp.transpose(boxes_t, (0, 2, 1))
  return scores, labels, boxes


# speedup vs baseline: 1.0452x; 1.0452x over previous
import functools

import jax
import jax.numpy as jnp
from jax import lax
from jax.experimental import pallas as pl
from jax.experimental.pallas import tpu as pltpu
from jax.experimental.pallas import tpu_sc as plsc

B, N, C = 8, 20000, 91
L = 16
NC, NS = 2, 16
NW = NC * NS
ST = 40
NSC = ST * 128
UPW = 2
CA, CB = 49, 42
OFFS = (0, 16, 32, 48, 64, 75)
ASTR = 96
BN = 1280
NTC = N - NSC


def _permute(g, idx):
  dn = lax.GatherDimensionNumbers(offset_dims=(), collapsed_slice_dims=(0,),
                                  start_index_map=(0,))
  return lax.gather(g, idx[:, None], dn, (1,),
                    mode=lax.GatherScatterMode.PROMISE_IN_BOUNDS)


def _sc_body(lg_hbm, pr_hbm, out_hbm,
             b0_v, b1_v, acc_v, sco_v, pr_v, at_v, semA, semB, semW):
  w = lax.axis_index("s") * NC + lax.axis_index("c")
  active = w * UPW < ST

  pltpu.sync_copy(pr_hbm.at[:, :], pr_v)

  def tab_img(img, carry):
    avecs = [1.0 + jnp.exp(-pr_v[img, pl.ds(off, L)]) for off in OFFS]

    def tab_lane(l, carry2):
      bl = jnp.broadcast_to(l, (L,))
      for j, off in enumerate(OFFS):
        at_v[pl.ds((img * ASTR + off + l) * L, L)] = _permute(avecs[j], bl)
      return carry2

    lax.fori_loop(0, L, tab_lane, 0)
    return carry

  lax.fori_loop(0, B, tab_img, 0)

  def issue(tc, buf, nclass, c0, sem):
    pltpu.async_copy(
        lg_hbm.at[pl.ds(c0, nclass), :, pl.ds(tc * 128, 128)], buf, sem)

  def wait_in(buf, nclass, sem):
    pltpu.make_async_copy(
        lg_hbm.at[pl.ds(0, nclass), :, pl.ds(0, 128)], buf, sem).wait()

  @pl.when(active)
  def _prologue():
    issue(w * UPW, b0_v, CA, 0, semA)
    issue(w * UPW, b1_v, CB, CA, semB)

  NA = 8
  inf8 = (jnp.full((L,), jnp.inf, jnp.float32),) * NA

  def make_cbody(buf, img, cbase):
    def cbody(c, accs):
      sp = at_v[pl.ds((img * ASTR + cbase + c) * L, L)]
      out = []
      for i in range(NA):
        x = buf[c, img, pl.ds(i * L, L)]
        out.append(jnp.minimum(accs[i], sp * jnp.exp(-x) + sp))
      return tuple(out)
    return cbody

  def compute_a(carry_unused):
    def grp(img, carry):
      accs = lax.fori_loop(0, CA, make_cbody(b0_v, img, 0), inf8, unroll=1)
      for i in range(NA):
        acc_v[pl.ds(img * 128 + i * L, L)] = accs[i]
      return carry
    lax.fori_loop(0, B, grp, 0)

  def compute_b(p):
    def grp(img, carry):
      init = tuple(acc_v[pl.ds(img * 128 + i * L, L)] for i in range(NA))
      accs = lax.fori_loop(0, CB, make_cbody(b1_v, img, CA), init, unroll=1)
      for i in range(NA):
        sco_v[p, img, pl.ds(i * L, L)] = 1.0 / accs[i]
      return carry
    lax.fori_loop(0, B, grp, 0)

  def ubody(k, carry):
    tc = w * UPW + k
    p = k & 1
    wait_in(b0_v, CA, semA)
    compute_a(None)

    @pl.when(k + 1 < UPW)
    def _ia():
      issue(tc + 1, b0_v, CA, 0, semA)

    wait_in(b1_v, CB, semB)
    compute_b(p)
    pltpu.async_copy(sco_v.at[p], out_hbm.at[:, pl.ds(tc * 128, 128)], semW)

    @pl.when(k + 1 < UPW)
    def _ib():
      issue(tc + 1, b1_v, CB, CA, semB)

    return carry

  @pl.when(active)
  def _run():
    lax.fori_loop(0, UPW, ubody, 0)
    for _ in range(UPW):
      pltpu.make_async_copy(sco_v.at[0], out_hbm.at[:, pl.ds(0, 128)],
                            semW).wait()


_sc_scores = functools.partial(
    pl.kernel,
    out_type=jax.ShapeDtypeStruct((B, NSC), jnp.float32),
    mesh=plsc.VectorSubcoreMesh(core_axis_name="c", subcore_axis_name="s",
                                num_cores=NC, num_subcores=NS),
    scratch_types=[
        pltpu.VMEM((CA, B, 128), jnp.float32),
        pltpu.VMEM((CB, B, 128), jnp.float32),
        pltpu.VMEM((B * 128,), jnp.float32),
        pltpu.VMEM((2, B, 128), jnp.float32),
        pltpu.VMEM((B, 128), jnp.float32),
        pltpu.VMEM((B * ASTR * L,), jnp.float32),
        pltpu.SemaphoreType.DMA,
        pltpu.SemaphoreType.DMA,
        pltpu.SemaphoreType.DMA,
    ],
    compiler_params=pltpu.CompilerParams(use_tc_tiling_on_sc=True))(_sc_body)


NG = NTC // BN + 1
BB = 128 * (-(-N // (128 * NG)))


def _tc_main_body(ts_ref, prt_ref, lg_ref, bx_ref,
                  sco_ref, box_ref, lab_ref):
  x = lg_ref[...]
  a = 1.0 + jnp.exp(-prt_ref[...][:C])
  acc = jnp.min(a[:, :, None] * jnp.exp(-x) + a[:, :, None], axis=0)
  sco_ref[...] = 1.0 / acc
  xb = bx_ref[...]
  ts = ts_ref[...].astype(jnp.float32)
  hh = ts[0][:, None, None]
  ww = ts[1][:, None, None]
  coord = lax.broadcasted_iota(jnp.int32, (B, 4, BB), 1)
  half = jnp.where(coord >= 2, 0.5, -0.5)
  cxy = jnp.concatenate([xb[:, 0:2], xb[:, 0:2]], axis=1)
  wh = jnp.concatenate([xb[:, 2:4], xb[:, 2:4]], axis=1)
  scale = jnp.where(coord % 2 == 0, ww, hh)
  box_ref[...] = (cxy + half * wh) * scale
  lab_ref[...] = jnp.ones((B, BB), jnp.int32)


def _tc_main(lgt, prt, bxt, tst):
  return pl.pallas_call(
      _tc_main_body,
      grid=(NG,),
      in_specs=[
          pl.BlockSpec((2, B), lambda j: (0, 0)),
          pl.BlockSpec((128, B), lambda j: (0, 0)),
          pl.BlockSpec((C, B, BN), lambda j: (0, 0, j + NSC // BN)),
          pl.BlockSpec((B, 4, BB), lambda j: (0, 0, j)),
      ],
      out_specs=[
          pl.BlockSpec((B, BN), lambda j: (0, j)),
          pl.BlockSpec((B, 4, BB), lambda j: (0, 0, j)),
          pl.BlockSpec((B, BB), lambda j: (0, j)),
      ],
      out_shape=[
          jax.ShapeDtypeStruct((B, NTC), jnp.float32),
          jax.ShapeDtypeStruct((B, 4, N), jnp.float32),
          jax.ShapeDtypeStruct((B, N), jnp.int32),
      ],
  )(tst, prt, lgt, bxt)


def kernel(pred_logits, pred_boxes, presence_logit_dec,
           target_sizes_boxes, target_sizes_masks):
  del target_sizes_masks
  lgt = jnp.transpose(pred_logits, (2, 0, 1))
  bxt = jnp.transpose(pred_boxes, (0, 2, 1))
  tst = jnp.transpose(target_sizes_boxes, (1, 0))
  pr_pad = jnp.pad(presence_logit_dec, ((0, 0), (0, 128 - C)))
  prt = jnp.transpose(pr_pad, (1, 0))
  sc_part = _sc_scores(lgt, pr_pad)
  tc_part, boxes_t, labels = _tc_main(lgt, prt, bxt, tst)
  scores = jnp.concatenate([sc_part, tc_part], axis=1)
  boxes = jnp.transpose(boxes_t, (0, 2, 1))
  return scores, labels, boxes


# speedup vs baseline: 1.0600x; 1.0142x over previous
import functools

import jax
import jax.numpy as jnp
from jax import lax
from jax.experimental import pallas as pl
from jax.experimental.pallas import tpu as pltpu
from jax.experimental.pallas import tpu_sc as plsc

B, N, C = 8, 20000, 91
L = 16
NC, NS = 2, 16
NW = NC * NS
ST = 40
NSC = ST * 128
UPW = 2
CA, CB = 49, 42
OFFS = (0, 16, 32, 48, 64, 75)
ASTR = 96
BN = 1024
NTC = N - NSC


def _permute(g, idx):
  dn = lax.GatherDimensionNumbers(offset_dims=(), collapsed_slice_dims=(0,),
                                  start_index_map=(0,))
  return lax.gather(g, idx[:, None], dn, (1,),
                    mode=lax.GatherScatterMode.PROMISE_IN_BOUNDS)


def _sc_body(lg_hbm, pr_hbm, out_hbm,
             b0_v, b1_v, acc_v, sco_v, pr_v, at_v, semA, semB, semW):
  w = lax.axis_index("s") * NC + lax.axis_index("c")
  active = w * UPW < ST

  pltpu.sync_copy(pr_hbm.at[:, :], pr_v)

  def tab_img(img, carry):
    avecs = [1.0 + jnp.exp(-pr_v[img, pl.ds(off, L)]) for off in OFFS]

    def tab_lane(l, carry2):
      bl = jnp.broadcast_to(l, (L,))
      for j, off in enumerate(OFFS):
        at_v[pl.ds((img * ASTR + off + l) * L, L)] = _permute(avecs[j], bl)
      return carry2

    lax.fori_loop(0, L, tab_lane, 0)
    return carry

  lax.fori_loop(0, B, tab_img, 0)

  def issue(tc, buf, nclass, c0, sem):
    pltpu.async_copy(
        lg_hbm.at[pl.ds(c0, nclass), :, pl.ds(tc * 128, 128)], buf, sem)

  def wait_in(buf, nclass, sem):
    pltpu.make_async_copy(
        lg_hbm.at[pl.ds(0, nclass), :, pl.ds(0, 128)], buf, sem).wait()

  @pl.when(active)
  def _prologue():
    issue(w * UPW, b0_v, CA, 0, semA)
    issue(w * UPW, b1_v, CB, CA, semB)

  NA = 8
  inf8 = (jnp.full((L,), jnp.inf, jnp.float32),) * NA

  def make_cbody(buf, img, cbase):
    def cbody(c, accs):
      sp = at_v[pl.ds((img * ASTR + cbase + c) * L, L)]
      out = []
      for i in range(NA):
        x = buf[c, img, pl.ds(i * L, L)]
        out.append(jnp.minimum(accs[i], sp * jnp.exp(-x) + sp))
      return tuple(out)
    return cbody

  def compute_a(carry_unused):
    def grp(img, carry):
      accs = lax.fori_loop(0, CA, make_cbody(b0_v, img, 0), inf8, unroll=1)
      for i in range(NA):
        acc_v[pl.ds(img * 128 + i * L, L)] = accs[i]
      return carry
    lax.fori_loop(0, B, grp, 0)

  def compute_b(p):
    def grp(img, carry):
      init = tuple(acc_v[pl.ds(img * 128 + i * L, L)] for i in range(NA))
      accs = lax.fori_loop(0, CB, make_cbody(b1_v, img, CA), init, unroll=1)
      for i in range(NA):
        sco_v[p, img, pl.ds(i * L, L)] = 1.0 / accs[i]
      return carry
    lax.fori_loop(0, B, grp, 0)

  def ubody(k, carry):
    tc = w * UPW + k
    p = k & 1
    wait_in(b0_v, CA, semA)
    compute_a(None)

    @pl.when(k + 1 < UPW)
    def _ia():
      issue(tc + 1, b0_v, CA, 0, semA)

    wait_in(b1_v, CB, semB)
    compute_b(p)
    pltpu.async_copy(sco_v.at[p], out_hbm.at[:, pl.ds(tc * 128, 128)], semW)

    @pl.when(k + 1 < UPW)
    def _ib():
      issue(tc + 1, b1_v, CB, CA, semB)

    return carry

  @pl.when(active)
  def _run():
    lax.fori_loop(0, UPW, ubody, 0)
    for _ in range(UPW):
      pltpu.make_async_copy(sco_v.at[0], out_hbm.at[:, pl.ds(0, 128)],
                            semW).wait()


_sc_scores = functools.partial(
    pl.kernel,
    out_type=jax.ShapeDtypeStruct((B, NSC), jnp.float32),
    mesh=plsc.VectorSubcoreMesh(core_axis_name="c", subcore_axis_name="s",
                                num_cores=NC, num_subcores=NS),
    scratch_types=[
        pltpu.VMEM((CA, B, 128), jnp.float32),
        pltpu.VMEM((CB, B, 128), jnp.float32),
        pltpu.VMEM((B * 128,), jnp.float32),
        pltpu.VMEM((2, B, 128), jnp.float32),
        pltpu.VMEM((B, 128), jnp.float32),
        pltpu.VMEM((B * ASTR * L,), jnp.float32),
        pltpu.SemaphoreType.DMA,
        pltpu.SemaphoreType.DMA,
        pltpu.SemaphoreType.DMA,
    ],
    compiler_params=pltpu.CompilerParams(use_tc_tiling_on_sc=True))(_sc_body)


NG = NTC // BN + 1
BB = 128 * (-(-N // (128 * NG)))


def _tc_main_body(ts_ref, prt_ref, lg_ref, bx_ref,
                  sco_ref, box_ref, lab_ref):
  x = lg_ref[...]
  a = 1.0 + jnp.exp(-prt_ref[...][:C])
  acc = jnp.min(a[:, :, None] * jnp.exp(-x) + a[:, :, None], axis=0)
  sco_ref[...] = 1.0 / acc
  xb = bx_ref[...]
  ts = ts_ref[...].astype(jnp.float32)
  hh = ts[0][:, None, None]
  ww = ts[1][:, None, None]
  coord = lax.broadcasted_iota(jnp.int32, (B, 4, BB), 1)
  half = jnp.where(coord >= 2, 0.5, -0.5)
  cxy = jnp.concatenate([xb[:, 0:2], xb[:, 0:2]], axis=1)
  wh = jnp.concatenate([xb[:, 2:4], xb[:, 2:4]], axis=1)
  scale = jnp.where(coord % 2 == 0, ww, hh)
  box_ref[...] = (cxy + half * wh) * scale
  lab_ref[...] = jnp.ones((B, BB), jnp.int32)


def _tc_main(lgt, prt, bxt, tst):
  return pl.pallas_call(
      _tc_main_body,
      grid=(NG,),
      in_specs=[
          pl.BlockSpec((2, B), lambda j: (0, 0)),
          pl.BlockSpec((128, B), lambda j: (0, 0)),
          pl.BlockSpec((C, B, BN), lambda j: (0, 0, j + NSC // BN)),
          pl.BlockSpec((B, 4, BB), lambda j: (0, 0, j)),
      ],
      out_specs=[
          pl.BlockSpec((B, BN), lambda j: (0, j)),
          pl.BlockSpec((B, 4, BB), lambda j: (0, 0, j)),
          pl.BlockSpec((B, BB), lambda j: (0, j)),
      ],
      out_shape=[
          jax.ShapeDtypeStruct((B, NTC), jnp.float32),
          jax.ShapeDtypeStruct((B, 4, N), jnp.float32),
          jax.ShapeDtypeStruct((B, N), jnp.int32),
      ],
  )(tst, prt, lgt, bxt)


def kernel(pred_logits, pred_boxes, presence_logit_dec,
           target_sizes_boxes, target_sizes_masks):
  del target_sizes_masks
  lgt = jnp.transpose(pred_logits, (2, 0, 1))
  bxt = jnp.transpose(pred_boxes, (0, 2, 1))
  tst = jnp.transpose(target_sizes_boxes, (1, 0))
  pr_pad = jnp.pad(presence_logit_dec, ((0, 0), (0, 128 - C)))
  prt = jnp.transpose(pr_pad, (1, 0))
  sc_part = _sc_scores(lgt, pr_pad)
  tc_part, boxes_t, labels = _tc_main(lgt, prt, bxt, tst)
  scores = jnp.concatenate([sc_part, tc_part], axis=1)
  boxes = jnp.transpose(boxes_t, (0, 2, 1))
  return scores, labels, boxes
